# baseline (speedup 1.0000x reference)
import functools

import jax
import jax.numpy as jnp
from jax import lax
from jax.experimental import pallas as pl
from jax.experimental.pallas import tpu as pltpu
from jax.experimental.pallas import tpu_sc as plsc

_NC = 2
_NS = 16
_NW = _NC * _NS

_B = 128
_KP = 512
_D = 512
_R = _B * _KP
_L = 16
_DC = _D // _L
_CH = 64
_NCH = _KP // _CH
_U = 8
_TPW = 2 * _B // _NW

_T = 0.07
_INV_COUNT = 1.0 / (2 * _B)


def _sc_gather_dot(mem1, mem2, idxf, v1, v2):
    mesh = plsc.VectorSubcoreMesh(core_axis_name="c", subcore_axis_name="s")

    @functools.partial(
        pl.kernel,
        mesh=mesh,
        compiler_params=pltpu.CompilerParams(use_tc_tiling_on_sc=False),
        out_type=(
            jax.ShapeDtypeStruct((2, _R, _L), jnp.float32),
            jax.ShapeDtypeStruct((2, _R, _L), jnp.float32),
        ),
        scratch_types=[
            pltpu.VMEM((_KP,), jnp.int32),
            pltpu.VMEM((_D,), jnp.float32),
            pltpu.VMEM((_D,), jnp.float32),
            pltpu.VMEM((_CH, _D), jnp.float32),
            pltpu.VMEM((_CH, _D), jnp.float32),
            pltpu.VMEM((_KP, _L), jnp.float32),
            pltpu.VMEM((_KP, _L), jnp.float32),
            pltpu.SemaphoreType.DMA,
            pltpu.SemaphoreType.DMA,
        ],
    )
    def k(m1, m2, idx_hbm, v1h, v2h, o1, o2,
          idx_v, v1_v, v2_v, buf_a, buf_b, p1_v, p2_v, sem_a, sem_b):
        wid = lax.axis_index("s") * _NC + lax.axis_index("c")
        bank = wid // _NS
        b0 = (wid % _NS) * _TPW

        def compute(buf, pbase):
            def group(g, carry):
                r0 = g * _U
                a1 = [jnp.zeros((_L,), jnp.float32) for _ in range(_U)]
                a2 = [jnp.zeros((_L,), jnp.float32) for _ in range(_U)]
                for c in range(_DC):
                    v1c = v1_v[pl.ds(c * _L, _L)]
                    v2c = v2_v[pl.ds(c * _L, _L)]
                    for u in range(_U):
                        rv = buf[r0 + u, pl.ds(c * _L, _L)]
                        a1[u] += rv * v1c
                        a2[u] += rv * v2c
                for u in range(_U):
                    p1_v[pbase + r0 + u, :] = a1[u]
                    p2_v[pbase + r0 + u, :] = a2[u]
                return carry
            lax.fori_loop(0, _CH // _U, group, 0)

        def gather(table, c, buf, sem):
            pltpu.async_copy(table.at[idx_v.at[pl.ds(c * _CH, _CH)]], buf, sem)

        def wait(table, buf, sem):
            pltpu.make_async_copy(table.at[idx_v.at[pl.ds(0, _CH)]], buf,
                                  sem).wait()

        def do_bank(table, q):
            def task(t, carry):
                b = b0 + t
                pltpu.sync_copy(v1h.at[b], v1_v)
                pltpu.sync_copy(v2h.at[b], v2_v)
                pltpu.sync_copy(idx_hbm.at[q, pl.ds(b * _KP, _KP)], idx_v)
                gather(table, 0, buf_a, sem_a)

                def cpair(c2, carry2):
                    c0 = c2 * 2
                    gather(table, c0 + 1, buf_b, sem_b)
                    wait(table, buf_a, sem_a)
                    compute(buf_a, c0 * _CH)
                    gather(table, jnp.minimum(c0 + 2, _NCH - 1), buf_a, sem_a)
                    wait(table, buf_b, sem_b)
                    compute(buf_b, (c0 + 1) * _CH)
                    return carry2
                lax.fori_loop(0, _NCH // 2, cpair, 0)
                wait(table, buf_a, sem_a)

                pltpu.sync_copy(p1_v, o1.at[q, pl.ds(b * _KP, _KP)])
                pltpu.sync_copy(p2_v, o2.at[q, pl.ds(b * _KP, _KP)])
                return carry
            lax.fori_loop(0, _TPW, task, 0)

        @pl.when(bank == 0)
        def _():
            do_bank(m1, 0)

        @pl.when(bank == 1)
        def _():
            do_bank(m2, 1)

    return k(mem1, mem2, idxf, v1, v2)


def _tc_loss_body(p1_ref, p2_ref, out_ref):
    b = pl.program_id(0)
    q1 = p1_ref[...]
    q2 = p2_ref[...]
    a11 = jnp.sum(q1[0, 0], axis=1, keepdims=True)
    a12 = jnp.sum(q1[1, 0], axis=1, keepdims=True)
    a21 = jnp.sum(q2[0, 0], axis=1, keepdims=True)
    a22 = jnp.sum(q2[1, 0], axis=1, keepdims=True)
    adc = jnp.concatenate(
        [jnp.concatenate([a11, a12], axis=0),
         jnp.concatenate([a21, a22], axis=0)], axis=1) / _T
    m = jnp.max(adc, axis=0, keepdims=True)
    lse = m + jnp.log(jnp.sum(jnp.exp(adc - m), axis=0, keepdims=True))
    row = lax.broadcasted_iota(jnp.int32, adc.shape, 0)
    pos_mask = (row == 0) | (row == _KP)
    pos = jnp.sum(jnp.where(pos_mask, adc, 0.0), axis=0, keepdims=True)
    contrib = jnp.sum(pos * 0.5 - lse)
    prev = jnp.where(b == 0, 0.0, out_ref[0, 0])
    acc = prev + contrib
    out_ref[0, 0] = jnp.where(b == _B - 1, -acc * _INV_COUNT, acc)


def _tc_loss(pp1, pp2):
    out = pl.pallas_call(
        _tc_loss_body,
        grid=(_B,),
        in_specs=[
            pl.BlockSpec((2, 1, _KP, _L), lambda b: (0, b, 0, 0)),
            pl.BlockSpec((2, 1, _KP, _L), lambda b: (0, b, 0, 0)),
        ],
        out_specs=pl.BlockSpec((1, 1), lambda b: (0, 0),
                               memory_space=pltpu.SMEM),
        out_shape=jax.ShapeDtypeStruct((1, 1), jnp.float32),
    )(pp1, pp2)
    return out[0, 0]


def kernel(v1, y1, v2, y2, idx1, idx2, memory_v1, memory_v2):
    idxf = jnp.stack([idx1.reshape(-1), idx2.reshape(-1)])
    p1, p2 = _sc_gather_dot(memory_v1, memory_v2, idxf, v1, v2)
    pp1 = p1.reshape(2, _B, _KP, _L)
    pp2 = p2.reshape(2, _B, _KP, _L)
    return _tc_loss(pp1, pp2)
